# baseline (speedup 1.0000x reference)
import functools

import numpy as np
import jax
import jax.numpy as jnp
from jax import lax
from jax.experimental import pallas as pl
from jax.experimental.pallas import tpu as pltpu
from jax.experimental.pallas import tpu_sc as plsc

_C = 64
_B, _T = 1024, 200
_NW = 32
_TOK = _B * _T
_PER_W = _TOK // _NW
_CH = 400
_NCH = _PER_W // _CH
_GPC = _CH // 16


def _sin_pe(T, C):
    pos = np.arange(T, dtype=np.float32)[:, None]
    div = np.exp(np.arange(0, C, 2, dtype=np.float32) * (-np.log(10000.0) / C))
    pe = np.zeros((T, C), dtype=np.float32)
    pe[:, 0::2] = np.sin(pos * div)
    pe[:, 1::2] = np.cos(pos * div)
    return pe


@functools.partial(
    pl.kernel,
    mesh=plsc.VectorSubcoreMesh(core_axis_name="c", subcore_axis_name="s"),
    out_type=jax.ShapeDtypeStruct((_TOK, _C), jnp.float32),
    scratch_types=[
        pltpu.VMEM((_PER_W,), jnp.int32),
        pltpu.VMEM((_T, _C), jnp.float32),
        pltpu.VMEM((_CH, _C), jnp.float32),
        pltpu.SemaphoreType.DMA,
        pltpu.SemaphoreType.DMA,
    ],
    compiler_params=pltpu.CompilerParams(use_tc_tiling_on_sc=True),
)
def _emb(idx_hbm, table_hbm, pe_hbm, out_hbm, idx_v, pe_v, buf, gsem, ssem):
    cid = lax.axis_index("c")
    sid = lax.axis_index("s")
    wid = sid * 2 + cid
    base = pl.multiple_of(wid * _PER_W, _PER_W)
    pltpu.sync_copy(idx_hbm.at[pl.ds(base, _PER_W)], idx_v)
    pltpu.sync_copy(pe_hbm, pe_v)

    def chunk(c, carry):
        def grp(g, carry2):
            vec = idx_v[pl.ds(c * _CH + g * 16, 16)]
            for l in range(16):
                r = vec[l]
                pltpu.async_copy(
                    table_hbm.at[r // 8, r % 8],
                    buf.at[g * 16 + l],
                    gsem,
                )
            return carry2

        lax.fori_loop(0, _GPC, grp, 0)
        pltpu.make_async_copy(
            out_hbm.at[pl.ds(0, _CH)], buf, gsem
        ).wait()

        def add_pe(tt, carry2):
            for dt in range(2):
                t = tt * 2 + dt
                for v in range(_C // 16):
                    sl = pl.ds(v * 16, 16)
                    p = pe_v[t, sl]
                    for rep in range(_CH // _T):
                        r = rep * _T + t
                        buf[r, sl] = buf[r, sl] + p
            return carry2

        lax.fori_loop(0, _T // 2, add_pe, 0)
        pltpu.async_copy(
            buf,
            out_hbm.at[pl.ds(pl.multiple_of(base + c * _CH, _CH), _CH)],
            ssem,
        ).wait()
        return carry

    lax.fori_loop(0, _NCH, chunk, 0)


def kernel(x, table):
    idx = x.reshape(_TOK).astype(jnp.int32)
    table3 = table.reshape(125000, 8, _C)
    pe = jnp.asarray(_sin_pe(_T, _C))
    out = _emb(idx, table3, pe)
    return out.reshape(_B, _T, _C)


# speedup vs baseline: 1.0606x; 1.0606x over previous
import functools

import numpy as np
import jax
import jax.numpy as jnp
from jax import lax
from jax.experimental import pallas as pl
from jax.experimental.pallas import tpu as pltpu
from jax.experimental.pallas import tpu_sc as plsc

_C = 64
_B, _T = 1024, 200
_NW = 32
_TOK = _B * _T
_PER_W = _TOK // _NW
_CH = 400
_NCH = _PER_W // _CH
_GPC = _CH // 16


def _sin_pe(T, C):
    pos = np.arange(T, dtype=np.float32)[:, None]
    div = np.exp(np.arange(0, C, 2, dtype=np.float32) * (-np.log(10000.0) / C))
    pe = np.zeros((T, C), dtype=np.float32)
    pe[:, 0::2] = np.sin(pos * div)
    pe[:, 1::2] = np.cos(pos * div)
    return pe


@functools.partial(
    pl.kernel,
    mesh=plsc.VectorSubcoreMesh(core_axis_name="c", subcore_axis_name="s"),
    out_type=jax.ShapeDtypeStruct((_TOK, _C), jnp.float32),
    scratch_types=[
        pltpu.VMEM((_PER_W,), jnp.int32),
        pltpu.VMEM((_T, _C), jnp.float32),
        pltpu.VMEM((_CH, _C), jnp.float32),
        pltpu.SemaphoreType.DMA,
        pltpu.SemaphoreType.DMA,
    ],
    compiler_params=pltpu.CompilerParams(use_tc_tiling_on_sc=True),
)
def _emb(idx_hbm, table_hbm, pe_hbm, out_hbm, idx_v, pe_v, buf, gsem, ssem):
    cid = lax.axis_index("c")
    sid = lax.axis_index("s")
    wid = sid * 2 + cid
    base = pl.multiple_of(wid * _PER_W, _PER_W)
    pltpu.sync_copy(idx_hbm.at[pl.ds(base, _PER_W)], idx_v)
    pltpu.sync_copy(pe_hbm, pe_v)

    def chunk(c, carry):
        def grp(g, carry2):
            vec = idx_v[pl.ds(c * _CH + g * 16, 16)]
            for l in range(16):
                r = vec[l]
                pltpu.async_copy(
                    table_hbm.at[r],
                    buf.at[g * 16 + l],
                    gsem,
                )
            return carry2

        lax.fori_loop(0, _GPC, grp, 0)
        pltpu.make_async_copy(
            out_hbm.at[pl.ds(0, _CH)], buf, gsem
        ).wait()

        def add_pe(tt, carry2):
            for dt in range(2):
                t = tt * 2 + dt
                for v in range(_C // 16):
                    sl = pl.ds(v * 16, 16)
                    p = pe_v[t, sl]
                    for rep in range(_CH // _T):
                        r = rep * _T + t
                        buf[r, sl] = buf[r, sl] + p
            return carry2

        lax.fori_loop(0, _T // 2, add_pe, 0)
        pltpu.async_copy(
            buf,
            out_hbm.at[pl.ds(pl.multiple_of(base + c * _CH, _CH), _CH)],
            ssem,
        ).wait()
        return carry

    lax.fori_loop(0, _NCH, chunk, 0)


def kernel(x, table):
    idx = x.reshape(_TOK).astype(jnp.int32)
    table3 = lax.optimization_barrier(table.reshape(125000, 8, _C))
    table3 = table3.reshape(1000000, _C)
    pe = jnp.asarray(_sin_pe(_T, _C))
    out = _emb(idx, table3, pe)
    return out.reshape(_B, _T, _C)
